# baseline (speedup 1.0000x reference)
import jax
import jax.numpy as jnp
from jax import lax
from jax.experimental import pallas as pl
from jax.experimental.pallas import tpu as pltpu
from jax.experimental.pallas import tpu_sc as plsc

NUM_CORES = 2
NUM_SUBCORES = 16
LANES = 16
TILES = NUM_CORES * NUM_SUBCORES

_MESH = dict(core_axis_name="c", subcore_axis_name="s")
_CP = pltpu.CompilerParams(needs_layout_passes=False, use_tc_tiling_on_sc=True)


def _wid():
    return lax.axis_index("s") * NUM_CORES + lax.axis_index("c")


def _iota16():
    return lax.iota(jnp.int32, LANES)


def _make_k2(B, bpt, nchunk):
    nb16 = bpt // LANES

    @pl.kernel(
        out_type=(
            jax.ShapeDtypeStruct((B // 2, 128), jnp.float32),
            jax.ShapeDtypeStruct((B,), jnp.int32),
        ),
        mesh=plsc.VectorSubcoreMesh(**_MESH),
        scratch_types=[
            pltpu.VMEM((bpt,), jnp.int32),
            pltpu.VMEM((bpt,), jnp.int32),
            pltpu.VMEM((bpt,), jnp.int32),
            pltpu.VMEM((nchunk, 128), jnp.int32),
            pltpu.VMEM((bpt, 128), jnp.float32),
            pltpu.VMEM((bpt // 2, 128), jnp.float32),
            pltpu.SemaphoreType.DMA,
        ],
        compiler_params=_CP,
    )
    def k2(t_hbm, comb_hbm, x2_hbm, emb_ref, out_hbm, eout_hbm,
           t_v, cm_v, sel_v, pair_v, rows_v, x_v, sem):
        w = _wid()
        base = w * bpt
        xbase = pl.multiple_of(w * (bpt // 2), 8)
        pltpu.sync_copy(t_hbm.at[pl.ds(base, bpt)], t_v)
        pltpu.async_copy(x2_hbm.at[pl.ds(xbase, bpt // 2)], x_v, sem).wait()

        for c in range(nchunk):
            pltpu.async_copy(comb_hbm.at[t_v.at[pl.ds(c * 128, 128)]],
                             cm_v.at[pl.ds(c * 128, 128)], sem).wait()

        @pl.loop(0, nb16)
        def _(c):
            off = c * LANES
            cm16 = cm_v[pl.ds(off, LANES)]
            e16 = cm16 >> 1
            ic = cm16 & 1
            bglob = base + off + _iota16()
            safe = jnp.where(ic != 0, jnp.maximum(e16, 0), bglob & 0x3FFF)
            cm_v.at[pl.ds(off, LANES)][...] = e16
            sel_v.at[pl.ds(off, LANES)][...] = (ic << 1) | (safe & 1)
            ci = c // 8
            cl = (c % 8) * LANES
            pair_v.at[ci, pl.ds(cl, LANES)][...] = safe >> 1

        for c in range(nchunk):
            pltpu.async_copy(emb_ref.at[pair_v.at[c]],
                             rows_v.at[pl.ds(c * 128, 128)], sem).wait()

        @pl.loop(0, bpt)
        def _(r):
            sel = plsc.load_gather(sel_v, [jnp.full((LANES,), r, jnp.int32)])
            m = sel >= 2
            half = sel & 1
            rrow = jnp.full((LANES,), r, jnp.int32)
            xrow = jnp.full((LANES,), r // 2, jnp.int32)
            for l in range(4):
                col = l * LANES + _iota16()
                pulled = plsc.load_gather(rows_v, [rrow, half * 64 + col])
                plsc.store_scatter(
                    x_v, [xrow, (r % 2) * 64 + col], pulled, mask=m)

        pltpu.sync_copy(x_v, out_hbm.at[pl.ds(xbase, bpt // 2)])
        pltpu.sync_copy(cm_v, eout_hbm.at[pl.ds(base, bpt)])

    return k2


def _make_k3(B, own):
    nchunks = B // LANES
    cap = own + 2 * 128

    @pl.kernel(
        mesh=plsc.VectorSubcoreMesh(**_MESH),
        scratch_types=[
            pltpu.VMEM((B,), jnp.int32),
            pltpu.VMEM((own,), jnp.int32),
            pltpu.VMEM((cap,), jnp.int32),
            pltpu.VMEM((cap,), jnp.int32),
            pltpu.VMEM((128,), jnp.int32),
            pltpu.VMEM((128,), jnp.int32),
            pltpu.VMEM((128,), jnp.int32),
            pltpu.VMEM((128, 128), jnp.float32),
            pltpu.VMEM((128, 128), jnp.float32),
            pltpu.SemaphoreType.DMA,
        ],
        compiler_params=_CP,
    )
    def k3(eall_hbm, x2_hbm, emb_ref,
           e_v, tag_v, ws_v, wb_v, sp_v, sx_v, sb_v, pr_v, xr_v, sem):
        w = _wid()
        base = w * own
        pltpu.sync_copy(eall_hbm, e_v)

        @pl.loop(0, own // LANES)
        def _(c):
            tag_v.at[pl.ds(c * LANES, LANES)][...] = jnp.full(
                (LANES,), -1, jnp.int32)

        @pl.loop(0, nchunks)
        def _(c):
            e16 = e_v[pl.ds(c * LANES, LANES)]
            b16 = c * LANES + _iota16()
            mask = (e16 >= base) & (e16 < base + own)
            off = jnp.where(mask, e16 - base, 0)
            plsc.store_scatter(tag_v, [off], b16, mask=mask)

        for parity in (0, 1):
            def cbody(c, cnt):
                t16 = tag_v[pl.ds(c * LANES, LANES)]
                s16 = base + c * LANES + _iota16()
                m = (t16 >= 0) & ((s16 & 1) == parity)
                plsc.store_compressed(ws_v.at[pl.ds(cnt, LANES)], s16, mask=m)
                plsc.store_compressed(wb_v.at[pl.ds(cnt, LANES)], t16, mask=m)
                return cnt + jnp.max(plsc.all_reduce_population_count(m))

            cnt = lax.fori_loop(0, own // LANES, cbody, jnp.int32(0))

            @pl.when(cnt > 0)
            def _():
                lasts = plsc.load_gather(
                    ws_v, [jnp.full((LANES,), cnt - 1, jnp.int32)])
                lastb = plsc.load_gather(
                    wb_v, [jnp.full((LANES,), cnt - 1, jnp.int32)])
                for k in range(8):
                    ws_v.at[pl.ds(cnt + k * LANES, LANES)][...] = lasts
                    wb_v.at[pl.ds(cnt + k * LANES, LANES)][...] = lastb

                def sbody(g, _):
                    @pl.loop(0, 8)
                    def _(c):
                        sl = pl.ds(c * LANES, LANES)
                        s16 = ws_v[pl.ds(g * 128 + c * LANES, LANES)]
                        b16 = wb_v[pl.ds(g * 128 + c * LANES, LANES)]
                        sp_v.at[sl][...] = s16 >> 1
                        sx_v.at[sl][...] = b16 >> 1
                        sb_v.at[sl][...] = b16 & 1
                    pltpu.async_copy(emb_ref.at[sp_v], pr_v, sem).wait()
                    pltpu.async_copy(x2_hbm.at[sx_v], xr_v, sem).wait()

                    @pl.loop(0, 128)
                    def _(i):
                        irow = jnp.full((LANES,), i, jnp.int32)
                        bh = plsc.load_gather(sb_v, [irow])
                        for l in range(4):
                            col = l * LANES + _iota16()
                            v = plsc.load_gather(xr_v, [irow, bh * 64 + col])
                            plsc.store_scatter(
                                pr_v, [irow, parity * 64 + col], v)

                    pltpu.async_copy(pr_v, emb_ref.at[sp_v], sem).wait()
                    return 0

                lax.fori_loop(0, (cnt + 127) // 128, sbody, 0)

    return k3


def _make_kf(B, fpt):
    nchunks = B // LANES
    wpt = fpt // 4

    @pl.kernel(
        out_type=jax.ShapeDtypeStruct((wpt * TILES,), jnp.int32),
        mesh=plsc.VectorSubcoreMesh(**_MESH),
        scratch_types=[
            pltpu.VMEM((fpt,), jnp.int32),
            pltpu.VMEM((B,), jnp.int32),
            pltpu.VMEM((B,), jnp.int32),
            pltpu.VMEM((wpt,), jnp.int32),
        ],
        compiler_params=_CP,
    )
    def kf(comb_hbm, t_hbm, eall_hbm, ncw_hbm, c_v, t_v, e_v, w_v):
        w = _wid()
        base = w * fpt
        pltpu.sync_copy(comb_hbm.at[pl.ds(base, fpt)], c_v)
        pltpu.sync_copy(t_hbm, t_v)
        pltpu.sync_copy(eall_hbm, e_v)

        @pl.loop(0, fpt // LANES)
        def _(c):
            sl = pl.ds(c * LANES, LANES)
            c_v.at[sl][...] = c_v.at[sl][...] & 1

        ones = jnp.full((LANES,), 1, jnp.int32)

        @pl.loop(0, nchunks)
        def _(c):
            t16 = t_v[pl.ds(c * LANES, LANES)]
            e16 = e_v[pl.ds(c * LANES, LANES)]
            mask = (e16 != -1) & (t16 >= base) & (t16 < base + fpt)
            off = jnp.where(mask, t16 - base, 0)
            plsc.store_scatter(c_v, [off], ones, mask=mask)

        @pl.loop(0, wpt // LANES)
        def _(j):
            i0 = (j * LANES + _iota16()) * 4
            wd = (plsc.load_gather(c_v, [i0])
                  | (plsc.load_gather(c_v, [i0 + 1]) << 8)
                  | (plsc.load_gather(c_v, [i0 + 2]) << 16)
                  | (plsc.load_gather(c_v, [i0 + 3]) << 24))
            w_v.at[pl.ds(j * LANES, LANES)][...] = wd

        pltpu.sync_copy(w_v, ncw_hbm.at[pl.ds(w * wpt, wpt)])

    return kf


def kernel(x, target_id, emb, emb_idx, cached_nodes):
    B, D = x.shape
    num_cache = emb.shape[0]
    num_emb = cached_nodes.shape[0]

    bpt = B // TILES
    nchunk = bpt // 128
    own = ((num_cache + TILES - 1) // TILES + 15) // 16 * 16
    fpt = ((num_emb + TILES - 1) // TILES + 63) // 64 * 64
    nflags_p = fpt * TILES

    comb = (emb_idx << 1) | cached_nodes.astype(jnp.int32)
    comb_p = jnp.pad(comb, (0, nflags_p - num_emb))
    x2 = x.reshape(B // 2, 2 * D)
    emb2 = emb.reshape(num_cache // 2, 2 * D)

    emb_ref = jax.new_ref(emb2)
    out2, eidx_all = _make_k2(B, bpt, nchunk)(target_id, comb_p, x2, emb_ref)
    ncw = _make_kf(B, fpt)(comb_p, target_id, eidx_all)
    _make_k3(B, own)(eidx_all, x2, emb_ref)
    new_emb = jax.freeze(emb_ref).reshape(num_cache, D)

    out = out2.reshape(B, D)
    new_cn = lax.bitcast_convert_type(
        ncw[:num_emb // 4], jnp.uint8).reshape(num_emb) != 0
    return out, new_emb, new_cn


# speedup vs baseline: 1.7281x; 1.7281x over previous
import jax
import jax.numpy as jnp
from jax import lax
from jax.experimental import pallas as pl
from jax.experimental.pallas import tpu as pltpu
from jax.experimental.pallas import tpu_sc as plsc

NUM_CORES = 2
NUM_SUBCORES = 16
LANES = 16
TILES = NUM_CORES * NUM_SUBCORES

_MESH = dict(core_axis_name="c", subcore_axis_name="s")
_CP = pltpu.CompilerParams(needs_layout_passes=False, use_tc_tiling_on_sc=True)


def _wid():
    return lax.axis_index("s") * NUM_CORES + lax.axis_index("c")


def _iota16():
    return lax.iota(jnp.int32, LANES)


def _pad128_body(in_ref, out_ref):
    tt = in_ref[...].T
    out_ref[...] = jnp.concatenate([tt, tt], axis=1)


def _tc_pad128(src_t, bn):
    d, n = src_t.shape
    return pl.pallas_call(
        _pad128_body,
        out_shape=jax.ShapeDtypeStruct((n, 128), jnp.float32),
        in_specs=[pl.BlockSpec((d, bn), lambda i: (0, i))],
        out_specs=pl.BlockSpec((bn, 128), lambda i: (i, 0)),
        grid=(pl.cdiv(n, bn),),
    )(src_t)


def _unpad_body(in_ref, out_ref):
    out_ref[...] = in_ref[:, 0:64].T


def _tc_unpad(tab, bn):
    n = tab.shape[0]
    return pl.pallas_call(
        _unpad_body,
        out_shape=jax.ShapeDtypeStruct((64, n), jnp.float32),
        in_specs=[pl.BlockSpec((bn, 128), lambda i: (i, 0))],
        out_specs=pl.BlockSpec((64, bn), lambda i: (0, i)),
        grid=(pl.cdiv(n, bn),),
    )(tab)


def _make_k2(B, bpt, nchunk):
    nb16 = bpt // LANES

    @pl.kernel(
        out_type=(
            jax.ShapeDtypeStruct((B, 64), jnp.float32),
            jax.ShapeDtypeStruct((B,), jnp.int32),
        ),
        mesh=plsc.VectorSubcoreMesh(**_MESH),
        scratch_types=[
            pltpu.VMEM((bpt,), jnp.int32),
            pltpu.VMEM((bpt,), jnp.int32),
            pltpu.VMEM((bpt,), jnp.int32),
            pltpu.VMEM((nchunk, 128), jnp.int32),
            pltpu.VMEM((128, 128), jnp.float32),
            pltpu.VMEM((bpt, 64), jnp.float32),
            pltpu.SemaphoreType.DMA,
        ],
        compiler_params=_CP,
    )
    def k2(t_hbm, comb_hbm, x64_hbm, emb_ref, out_hbm, eout_hbm,
           t_v, cm_v, ic_v, safe_v, rows_v, x_v, sem):
        w = _wid()
        base = pl.multiple_of(w * bpt, 8)
        pltpu.sync_copy(t_hbm.at[pl.ds(base, bpt)], t_v)
        pltpu.async_copy(x64_hbm.at[pl.ds(base, bpt)], x_v, sem).wait()

        for c in range(nchunk):
            pltpu.async_copy(comb_hbm.at[t_v.at[pl.ds(c * 128, 128)]],
                             cm_v.at[pl.ds(c * 128, 128)], sem).wait()

        @pl.loop(0, nb16)
        def _(c):
            off = c * LANES
            cm16 = cm_v[pl.ds(off, LANES)]
            e16 = cm16 >> 1
            ic = cm16 & 1
            bglob = base + off + _iota16()
            safe = jnp.where(ic != 0, jnp.maximum(e16, 0), bglob & 0x3FFF)
            cm_v.at[pl.ds(off, LANES)][...] = e16
            ic_v.at[pl.ds(off, LANES)][...] = ic
            ci = c // 8
            cl = (c % 8) * LANES
            safe_v.at[ci, pl.ds(cl, LANES)][...] = safe

        for c in range(nchunk):
            pltpu.async_copy(emb_ref.at[safe_v.at[c]], rows_v, sem).wait()

            @pl.loop(0, 128)
            def _(r):
                rg = c * 128 + r
                m = plsc.load_gather(
                    ic_v, [jnp.full((LANES,), rg, jnp.int32)]) != 0
                for l in range(4):
                    col = l * LANES + _iota16()
                    pulled = plsc.load_gather(
                        rows_v, [jnp.full((LANES,), r, jnp.int32), col])
                    plsc.store_scatter(
                        x_v, [jnp.full((LANES,), rg, jnp.int32), col],
                        pulled, mask=m)

        pltpu.sync_copy(x_v, out_hbm.at[pl.ds(base, bpt)])
        pltpu.sync_copy(cm_v, eout_hbm.at[pl.ds(base, bpt)])

    return k2


def _make_k3(B, own):
    nchunks = B // LANES
    cap = own + 2 * 128

    @pl.kernel(
        mesh=plsc.VectorSubcoreMesh(**_MESH),
        scratch_types=[
            pltpu.VMEM((B,), jnp.int32),
            pltpu.VMEM((own,), jnp.int32),
            pltpu.VMEM((cap,), jnp.int32),
            pltpu.VMEM((cap,), jnp.int32),
            pltpu.VMEM((128,), jnp.int32),
            pltpu.VMEM((128,), jnp.int32),
            pltpu.VMEM((128, 128), jnp.float32),
            pltpu.SemaphoreType.DMA,
        ],
        compiler_params=_CP,
    )
    def k3(eall_hbm, xp_hbm, emb_ref,
           e_v, tag_v, ws_v, wb_v, ss_v, sb_v, xr_v, sem):
        w = _wid()
        base = w * own
        pltpu.sync_copy(eall_hbm, e_v)

        @pl.loop(0, own // LANES)
        def _(c):
            tag_v.at[pl.ds(c * LANES, LANES)][...] = jnp.full(
                (LANES,), -1, jnp.int32)

        @pl.loop(0, nchunks)
        def _(c):
            e16 = e_v[pl.ds(c * LANES, LANES)]
            b16 = c * LANES + _iota16()
            mask = (e16 >= base) & (e16 < base + own)
            off = jnp.where(mask, e16 - base, 0)
            plsc.store_scatter(tag_v, [off], b16, mask=mask)

        def cbody(c, cnt):
            t16 = tag_v[pl.ds(c * LANES, LANES)]
            s16 = base + c * LANES + _iota16()
            m = t16 >= 0
            plsc.store_compressed(ws_v.at[pl.ds(cnt, LANES)], s16, mask=m)
            plsc.store_compressed(wb_v.at[pl.ds(cnt, LANES)], t16, mask=m)
            return cnt + jnp.max(plsc.all_reduce_population_count(m))

        cnt = lax.fori_loop(0, own // LANES, cbody, jnp.int32(0))

        @pl.when(cnt > 0)
        def _():
            lasts = plsc.load_gather(
                ws_v, [jnp.full((LANES,), cnt - 1, jnp.int32)])
            lastb = plsc.load_gather(
                wb_v, [jnp.full((LANES,), cnt - 1, jnp.int32)])
            for k in range(8):
                ws_v.at[pl.ds(cnt + k * LANES, LANES)][...] = lasts
                wb_v.at[pl.ds(cnt + k * LANES, LANES)][...] = lastb

            def sbody(g, _):
                @pl.loop(0, 8)
                def _(c):
                    sl = pl.ds(c * LANES, LANES)
                    ss_v.at[sl][...] = ws_v[pl.ds(g * 128 + c * LANES, LANES)]
                    sb_v.at[sl][...] = wb_v[pl.ds(g * 128 + c * LANES, LANES)]
                pltpu.async_copy(xp_hbm.at[sb_v], xr_v, sem).wait()
                pltpu.async_copy(xr_v, emb_ref.at[ss_v], sem).wait()
                return 0

            lax.fori_loop(0, (cnt + 127) // 128, sbody, 0)

    return k3


def _make_kf(B, fpt):
    nchunks = B // LANES

    @pl.kernel(
        out_type=jax.ShapeDtypeStruct((fpt * TILES,), jnp.int32),
        mesh=plsc.VectorSubcoreMesh(**_MESH),
        scratch_types=[
            pltpu.VMEM((fpt,), jnp.int32),
            pltpu.VMEM((B,), jnp.int32),
            pltpu.VMEM((B,), jnp.int32),
        ],
        compiler_params=_CP,
    )
    def kf(comb_hbm, t_hbm, eall_hbm, ncf_hbm, c_v, t_v, e_v):
        w = _wid()
        base = w * fpt
        pltpu.sync_copy(comb_hbm.at[pl.ds(base, fpt)], c_v)
        pltpu.sync_copy(t_hbm, t_v)
        pltpu.sync_copy(eall_hbm, e_v)

        @pl.loop(0, fpt // LANES)
        def _(c):
            sl = pl.ds(c * LANES, LANES)
            c_v.at[sl][...] = c_v.at[sl][...] & 1

        ones = jnp.full((LANES,), 1, jnp.int32)

        @pl.loop(0, nchunks)
        def _(c):
            t16 = t_v[pl.ds(c * LANES, LANES)]
            e16 = e_v[pl.ds(c * LANES, LANES)]
            mask = (e16 != -1) & (t16 >= base) & (t16 < base + fpt)
            off = jnp.where(mask, t16 - base, 0)
            plsc.store_scatter(c_v, [off], ones, mask=mask)

        pltpu.sync_copy(c_v, ncf_hbm.at[pl.ds(base, fpt)])

    return kf


def kernel(x, target_id, emb, emb_idx, cached_nodes):
    B, D = x.shape
    num_cache = emb.shape[0]
    num_emb = cached_nodes.shape[0]

    bpt = B // TILES
    nchunk = bpt // 128
    own = ((num_cache + TILES - 1) // TILES + 15) // 16 * 16
    fpt = ((num_emb + TILES - 1) // TILES + 63) // 64 * 64
    nflags_p = fpt * TILES

    comb = (emb_idx << 1) | cached_nodes.astype(jnp.int32)
    comb_p = jnp.pad(comb, (0, nflags_p - num_emb))

    embP = _tc_pad128(emb.T, 4096)
    xP = _tc_pad128(x.T, 4096)

    emb_ref = jax.new_ref(embP)
    out, eidx_all = _make_k2(B, bpt, nchunk)(target_id, comb_p, x, emb_ref)
    ncw = _make_kf(B, fpt)(comb_p, target_id, eidx_all)
    _make_k3(B, own)(eidx_all, xP, emb_ref)
    new_emb = _tc_unpad(jax.freeze(emb_ref), 4096).T

    new_cn = ncw[:num_emb] != 0
    return out, new_emb, new_cn
